# baseline (speedup 1.0000x reference)
import functools

import jax
import jax.numpy as jnp
from jax import lax
from jax.experimental import pallas as pl
from jax.experimental.pallas import tpu as pltpu
from jax.experimental.pallas import tpu_sc as plsc

R, C = 4096, 2048
NC, NS = 2, 16
NW = NC * NS
ROWS_PER_W = R // NW
G = 8
L = 16
CHUNKS = C // L


def _scan_body(x_hbm, out_hbm, buf):
    wid = lax.axis_index("s") * NC + lax.axis_index("c")
    base = wid * ROWS_PER_W

    def group_body(g, _):
        r0 = base + g * G
        pltpu.sync_copy(x_hbm.at[pl.ds(r0, G), :], buf)

        def row_body(r, _):
            def chunk_body(k, carry):
                v = buf[r, pl.ds(k * L, L)]
                buf[r, pl.ds(k * L, L)] = plsc.cumsum(v) + carry
                return carry + jnp.sum(v)

            lax.fori_loop(0, CHUNKS, chunk_body, jnp.float32(0.0), unroll=4)
            return 0

        lax.fori_loop(0, G, row_body, 0)
        pltpu.sync_copy(buf, out_hbm.at[pl.ds(r0, G), :])
        return 0

    lax.fori_loop(0, ROWS_PER_W // G, group_body, 0)


def kernel(x):
    mesh = plsc.VectorSubcoreMesh(
        core_axis_name="c", subcore_axis_name="s", num_cores=NC, num_subcores=NS
    )
    scan = functools.partial(
        pl.kernel,
        out_type=jax.ShapeDtypeStruct((R, C), jnp.float32),
        mesh=mesh,
        scratch_types=[pltpu.VMEM((G, C), jnp.float32)],
        compiler_params=pltpu.CompilerParams(needs_layout_passes=False),
    )(_scan_body)
    return scan(x)


# speedup vs baseline: 4.4536x; 4.4536x over previous
import functools

import jax
import jax.numpy as jnp
from jax import lax
from jax.experimental import pallas as pl
from jax.experimental.pallas import tpu as pltpu
from jax.experimental.pallas import tpu_sc as plsc

R, C = 4096, 2048
F = 1024
NC, NS = 2, 16
NW = NC * NS
ROWS_PER_W = F // NW
G = 8
NG = ROWS_PER_W // G
L = 16
CHUNKS = C // L


def _scan_rows(buf, idx_last):
    def row_body(r, _):
        def chunk_body(k, carry):
            v = buf[r, pl.ds(k * L, L)]
            s = plsc.cumsum(v)
            t = s.at[idx_last].get(mode="promise_in_bounds")
            buf[r, pl.ds(k * L, L)] = s + carry
            return carry + t

        lax.fori_loop(0, CHUNKS, chunk_body, jnp.zeros((L,), jnp.float32),
                      unroll=8)
        return 0

    lax.fori_loop(0, G, row_body, 0)


def _sc_body(x_hbm, out_hbm, buf_a, buf_b, isem_a, isem_b, osem_a, osem_b):
    wid = lax.axis_index("s") * NC + lax.axis_index("c")
    base = wid * ROWS_PER_W
    idx_last = jnp.full((L,), L - 1, jnp.int32)
    bufs = (buf_a, buf_b)
    isems = (isem_a, isem_b)
    osems = (osem_a, osem_b)

    def in_copy(g, s):
        return pltpu.make_async_copy(
            x_hbm.at[pl.ds(base + g * G, G), :], bufs[s], isems[s])

    def out_copy(g, s):
        return pltpu.make_async_copy(
            bufs[s], out_hbm.at[pl.ds(base + g * G, G), :], osems[s])

    in_copy(0, 0).start()
    for g in range(NG):
        s = g % 2
        in_copy(g, s).wait()
        if g + 1 < NG:
            if g >= 1:
                out_copy(g - 1, 1 - s).wait()
            in_copy(g + 1, 1 - s).start()
        _scan_rows(bufs[s], idx_last)
        out_copy(g, s).start()
    out_copy(NG - 2, (NG - 2) % 2).wait()
    out_copy(NG - 1, (NG - 1) % 2).wait()


def _sc_scan(x):
    mesh = plsc.VectorSubcoreMesh(
        core_axis_name="c", subcore_axis_name="s", num_cores=NC, num_subcores=NS
    )
    scan = functools.partial(
        pl.kernel,
        out_type=jax.ShapeDtypeStruct((F, C), jnp.float32),
        mesh=mesh,
        scratch_types=[
            pltpu.VMEM((G, C), jnp.float32),
            pltpu.VMEM((G, C), jnp.float32),
            pltpu.SemaphoreType.DMA,
            pltpu.SemaphoreType.DMA,
            pltpu.SemaphoreType.DMA,
            pltpu.SemaphoreType.DMA,
        ],
        compiler_params=pltpu.CompilerParams(needs_layout_passes=False),
    )(_sc_body)
    return scan(x)


BM = 256
SEG = 128
NSEG = C // SEG


def _tc_body(x_ref, o_ref):
    u = jnp.triu(jnp.ones((SEG, SEG), jnp.float32))
    running = jnp.zeros((BM, 1), jnp.float32)
    for t in range(NSEG):
        seg = x_ref[:, t * SEG:(t + 1) * SEG]
        within = jax.lax.dot(seg, u, precision=jax.lax.Precision.HIGHEST,
                             preferred_element_type=jnp.float32)
        o_ref[:, t * SEG:(t + 1) * SEG] = within + running
        running = running + within[:, SEG - 1:SEG]


def _tc_scan(x):
    nblk = (R - F) // BM
    return pl.pallas_call(
        _tc_body,
        grid=(nblk,),
        in_specs=[pl.BlockSpec((BM, C), lambda i: (F // BM + i, 0))],
        out_specs=pl.BlockSpec((BM, C), lambda i: (F // BM + i, 0)),
        out_shape=jax.ShapeDtypeStruct((R, C), jnp.float32),
        compiler_params=pltpu.CompilerParams(
            dimension_semantics=("arbitrary",)),
    )(x)


def kernel(x):
    sc_out = _sc_scan(x)
    tc_full = _tc_scan(x)
    return lax.dynamic_update_slice(tc_full, sc_out, (0, 0))


# speedup vs baseline: 5.1006x; 1.1453x over previous
import functools

import jax
import jax.numpy as jnp
from jax import lax
from jax.experimental import pallas as pl
from jax.experimental.pallas import tpu as pltpu
from jax.experimental.pallas import tpu_sc as plsc

R, C = 4096, 2048
F = 768
NC, NS = 2, 16
NW = NC * NS
ROWS_PER_W = F // NW
G = 8
NG = ROWS_PER_W // G
L = 16
CHUNKS = C // L


def _scan_rows(buf, idx_last):
    def row_body(r, _):
        def chunk_body(k, carry):
            v = buf[r, pl.ds(k * L, L)]
            s = plsc.cumsum(v)
            t = s.at[idx_last].get(mode="promise_in_bounds")
            buf[r, pl.ds(k * L, L)] = s + carry
            return carry + t

        lax.fori_loop(0, CHUNKS, chunk_body, jnp.zeros((L,), jnp.float32),
                      unroll=8)
        return 0

    lax.fori_loop(0, G, row_body, 0)


def _sc_body(x_hbm, out_hbm, buf_a, buf_b, isem_a, isem_b, osem_a, osem_b):
    wid = lax.axis_index("s") * NC + lax.axis_index("c")
    base = wid * ROWS_PER_W
    idx_last = jnp.full((L,), L - 1, jnp.int32)
    bufs = (buf_a, buf_b)
    isems = (isem_a, isem_b)
    osems = (osem_a, osem_b)

    def in_copy(g, s):
        return pltpu.make_async_copy(
            x_hbm.at[pl.ds(base + g * G, G), :], bufs[s], isems[s])

    def out_copy(g, s):
        return pltpu.make_async_copy(
            bufs[s], out_hbm.at[pl.ds(base + g * G, G), :], osems[s])

    in_copy(0, 0).start()
    for g in range(NG):
        s = g % 2
        in_copy(g, s).wait()
        if g + 1 < NG:
            if g >= 1:
                out_copy(g - 1, 1 - s).wait()
            in_copy(g + 1, 1 - s).start()
        _scan_rows(bufs[s], idx_last)
        out_copy(g, s).start()
    out_copy(NG - 2, (NG - 2) % 2).wait()
    out_copy(NG - 1, (NG - 1) % 2).wait()


def _sc_scan(x):
    mesh = plsc.VectorSubcoreMesh(
        core_axis_name="c", subcore_axis_name="s", num_cores=NC, num_subcores=NS
    )
    scan = functools.partial(
        pl.kernel,
        out_type=jax.ShapeDtypeStruct((F, C), jnp.float32),
        mesh=mesh,
        scratch_types=[
            pltpu.VMEM((G, C), jnp.float32),
            pltpu.VMEM((G, C), jnp.float32),
            pltpu.SemaphoreType.DMA,
            pltpu.SemaphoreType.DMA,
            pltpu.SemaphoreType.DMA,
            pltpu.SemaphoreType.DMA,
        ],
        compiler_params=pltpu.CompilerParams(needs_layout_passes=False),
    )(_sc_body)
    return scan(x)


BM = 256
SEG = 128
NSEG = C // SEG


def _tc_body(x_ref, o_ref):
    u = jnp.triu(jnp.ones((SEG, SEG), jnp.float32))
    running = jnp.zeros((BM, 1), jnp.float32)
    for t in range(NSEG):
        seg = x_ref[:, t * SEG:(t + 1) * SEG]
        within = jax.lax.dot(seg, u, preferred_element_type=jnp.float32)
        o_ref[:, t * SEG:(t + 1) * SEG] = within + running
        running = running + within[:, SEG - 1:SEG]


def _tc_scan(x):
    nblk = (R - F) // BM
    return pl.pallas_call(
        _tc_body,
        grid=(nblk,),
        in_specs=[pl.BlockSpec((BM, C), lambda i: (F // BM + i, 0))],
        out_specs=pl.BlockSpec((BM, C), lambda i: (F // BM + i, 0)),
        out_shape=jax.ShapeDtypeStruct((R, C), jnp.float32),
        compiler_params=pltpu.CompilerParams(
            dimension_semantics=("arbitrary",)),
    )(x)


def kernel(x):
    sc_out = _sc_scan(x)
    tc_full = _tc_scan(x)
    return lax.dynamic_update_slice(tc_full, sc_out, (0, 0))
